# baseline (device time: 20473 ns/iter reference)
import jax
import jax.numpy as jnp
from jax import lax
from jax.experimental import pallas as pl
from jax.experimental.pallas import tpu as pltpu

QSCALE = 127.0 / 4.5
QINV = 4.5 / 127.0

CH = 128
YC = 5
XC = 3
OVL = XC * CH
NOUT = YC + XC


def kernel(partial, resid, gamma):
    m, d = resid.shape

    def body(partial_ref, resid_ref, gamma_ref, out_ref,
             send_buf, other_buf, resid_vmem, out_stage,
             y_send_sems, y_recv_sems, x_send_sems, x_recv_sems,
             resid_sem, out_sems):
        my_x = lax.axis_index("x")
        my_y = lax.axis_index("y")
        my_z = lax.axis_index("z")
        y_peer = (my_x, 1 - my_y, my_z)
        x_nbr = (1 - my_x, my_y, my_z)

        resid_copy = pltpu.make_async_copy(resid_ref, resid_vmem, resid_sem)
        resid_copy.start()

        barrier_sem = pltpu.get_barrier_semaphore()
        for nbr in (y_peer, x_nbr):
            pl.semaphore_signal(
                barrier_sem, inc=1,
                device_id=nbr, device_id_type=pl.DeviceIdType.MESH,
            )
        pl.semaphore_wait(barrier_sem, 2)

        def y_off(c):
            if c < XC:
                return c * CH + my_x * (YC * CH)
            return OVL + (c - XC) * CH

        y_rdmas = []
        for c in range(YC):
            off = y_off(c)
            send_buf[pl.ds(c * CH, CH), :] = jnp.round(
                jnp.clip(
                    partial_ref[0, pl.ds(off, CH), :] * QSCALE, -127.0, 127.0
                )
            ).astype(jnp.int8)
            r = pltpu.make_async_remote_copy(
                src_ref=send_buf.at[pl.ds(c * CH, CH)],
                dst_ref=other_buf.at[pl.ds(c * CH, CH)],
                send_sem=y_send_sems.at[c],
                recv_sem=y_recv_sems.at[c],
                device_id=y_peer,
                device_id_type=pl.DeviceIdType.MESH,
            )
            r.start()
            y_rdmas.append(r)

        resid_copy.wait()

        out_dmas = []

        def compute_rows(slot, off):
            i = len(out_dmas)
            s = i % 2
            if i >= 2:
                out_dmas[i - 2].wait()
            rows = (
                partial_ref[0, pl.ds(off, CH), :]
                + other_buf[pl.ds(slot * CH, CH), :].astype(jnp.float32) * QINV
                + resid_vmem[pl.ds(off, CH), :]
            )
            inv = lax.rsqrt(jnp.mean(rows * rows, axis=-1, keepdims=True) + 1e-6)
            out_stage[s] = rows * inv * gamma_ref[...]
            w = pltpu.make_async_copy(
                out_stage.at[s], out_ref.at[pl.ds(off, CH)], out_sems.at[i]
            )
            w.start()
            out_dmas.append(w)

        x_rdmas = []
        for c in range(YC):
            off = y_off(c)
            y_rdmas[c].wait_recv()
            if c < XC:
                r = pltpu.make_async_remote_copy(
                    src_ref=other_buf.at[pl.ds(c * CH, CH)],
                    dst_ref=other_buf.at[pl.ds((YC + c) * CH, CH)],
                    send_sem=x_send_sems.at[c],
                    recv_sem=x_recv_sems.at[c],
                    device_id=x_nbr,
                    device_id_type=pl.DeviceIdType.MESH,
                )
                r.start()
                x_rdmas.append(r)
            compute_rows(c, off)

        for c in range(XC):
            other_off = c * CH + (1 - my_x) * (YC * CH)
            x_rdmas[c].wait_recv()
            compute_rows(YC + c, other_off)

        out_dmas[NOUT - 2].wait()
        out_dmas[NOUT - 1].wait()
        for c in range(YC):
            y_rdmas[c].wait_send()
        for c in range(XC):
            x_rdmas[c].wait_send()

    return pl.pallas_call(
        body,
        out_shape=jax.ShapeDtypeStruct((m, d), jnp.float32),
        in_specs=[
            pl.BlockSpec(memory_space=pltpu.VMEM),
            pl.BlockSpec(memory_space=pl.ANY),
            pl.BlockSpec(memory_space=pltpu.VMEM),
        ],
        out_specs=pl.BlockSpec(memory_space=pl.ANY),
        scratch_shapes=[
            pltpu.VMEM((YC * CH, d), jnp.int8),
            pltpu.VMEM(((YC + XC) * CH, d), jnp.int8),
            pltpu.VMEM((m, d), jnp.float32),
            pltpu.VMEM((2, CH, d), jnp.float32),
            pltpu.SemaphoreType.DMA((YC,)),
            pltpu.SemaphoreType.DMA((YC,)),
            pltpu.SemaphoreType.DMA((XC,)),
            pltpu.SemaphoreType.DMA((XC,)),
            pltpu.SemaphoreType.DMA,
            pltpu.SemaphoreType.DMA((NOUT,)),
        ],
        compiler_params=pltpu.CompilerParams(collective_id=0),
    )(partial, resid, gamma.reshape(1, d))


# device time: 19890 ns/iter; 1.0293x vs baseline; 1.0293x over previous
import jax
import jax.numpy as jnp
from jax import lax
from jax.experimental import pallas as pl
from jax.experimental.pallas import tpu as pltpu

QSCALE = 127.0 / 4.5
QINV = 4.5 / 127.0

CH = 64
QC = 4
Q = QC * CH


def kernel(partial, resid, gamma):
    m, d = resid.shape

    def body(partial_ref, resid_ref, gamma_ref, out_ref,
             send_buf, other_buf,
             y_send, y_recv, xq_send, xq_recv, zq_send, zq_recv,
             xd_send, xd_recv, zd_send, zd_recv):
        my_x = lax.axis_index("x")
        my_y = lax.axis_index("y")
        my_z = lax.axis_index("z")
        qz = my_z % 2
        pz = my_z + 1 - 2 * qz
        y_peer = (my_x, 1 - my_y, my_z)
        b_nbr = (1 - my_x, my_y, my_z)
        c_nbr = (my_x, my_y, pz)

        k_me = 2 * my_x + qz
        k_b = 2 * (1 - my_x) + qz
        k_c = 2 * my_x + (1 - qz)
        k_d = 2 * (1 - my_x) + (1 - qz)

        barrier_sem = pltpu.get_barrier_semaphore()
        for nbr in (y_peer, b_nbr, c_nbr):
            pl.semaphore_signal(
                barrier_sem, inc=1,
                device_id=nbr, device_id_type=pl.DeviceIdType.MESH,
            )
        pl.semaphore_wait(barrier_sem, 3)

        def rcopy(src_slot, dst_slot, send_sem, recv_sem, dev):
            return pltpu.make_async_remote_copy(
                src_ref=other_buf.at[pl.ds(src_slot * CH, CH)],
                dst_ref=other_buf.at[pl.ds(dst_slot * CH, CH)],
                send_sem=send_sem,
                recv_sem=recv_sem,
                device_id=dev,
                device_id_type=pl.DeviceIdType.MESH,
            )

        y_rdmas = []
        for c in range(QC):
            send_buf[pl.ds(c * CH, CH), :] = jnp.round(
                jnp.clip(
                    partial_ref[0, pl.ds(k_me * Q + c * CH, CH), :] * QSCALE,
                    -127.0, 127.0,
                )
            ).astype(jnp.int8)
            r = pltpu.make_async_remote_copy(
                src_ref=send_buf.at[pl.ds(c * CH, CH)],
                dst_ref=other_buf.at[pl.ds(c * CH, CH)],
                send_sem=y_send.at[c],
                recv_sem=y_recv.at[c],
                device_id=y_peer,
                device_id_type=pl.DeviceIdType.MESH,
            )
            r.start()
            y_rdmas.append(r)

        def compute_rows(slot, k, c):
            off = k * Q + c * CH
            rows = (
                partial_ref[0, pl.ds(off, CH), :]
                + other_buf[pl.ds(slot * CH, CH), :].astype(jnp.float32) * QINV
                + resid_ref[pl.ds(off, CH), :]
            )
            inv = lax.rsqrt(jnp.mean(rows * rows, axis=-1, keepdims=True) + 1e-6)
            out_ref[pl.ds(off, CH), :] = rows * inv * gamma_ref[...]

        xq_rdmas, zq_rdmas = [], []
        for c in range(QC):
            y_rdmas[c].wait_recv()
            r = rcopy(c, 4 + c, xq_send.at[c], xq_recv.at[c], b_nbr)
            r.start()
            xq_rdmas.append(r)
            r = rcopy(c, 8 + c, zq_send.at[c], zq_recv.at[c], c_nbr)
            r.start()
            zq_rdmas.append(r)
            compute_rows(c, k_me, c)

        xd_rdmas, zd_rdmas = [], []
        for c in range(QC):
            xq_rdmas[c].wait_recv()
            if c >= 2:
                r = rcopy(4 + c, 12 + c, zd_send.at[c - 2], zd_recv.at[c - 2],
                          c_nbr)
                r.start()
                zd_rdmas.append(r)
            compute_rows(4 + c, k_b, c)

            zq_rdmas[c].wait_recv()
            if c < 2:
                r = rcopy(8 + c, 12 + c, xd_send.at[c], xd_recv.at[c], b_nbr)
                r.start()
                xd_rdmas.append(r)
            compute_rows(8 + c, k_c, c)

        for c in range(2):
            xd_rdmas[c].wait_recv()
            compute_rows(12 + c, k_d, c)
        for c in range(2):
            zd_rdmas[c].wait_recv()
            compute_rows(14 + c, k_d, 2 + c)

        for r in y_rdmas + xq_rdmas + zq_rdmas + xd_rdmas + zd_rdmas:
            r.wait_send()

    return pl.pallas_call(
        body,
        out_shape=jax.ShapeDtypeStruct((m, d), jnp.float32),
        in_specs=[
            pl.BlockSpec(memory_space=pltpu.VMEM),
            pl.BlockSpec(memory_space=pltpu.VMEM),
            pl.BlockSpec(memory_space=pltpu.VMEM),
        ],
        out_specs=pl.BlockSpec(memory_space=pltpu.VMEM),
        scratch_shapes=[
            pltpu.VMEM((QC * CH, d), jnp.int8),
            pltpu.VMEM((16 * CH, d), jnp.int8),
            pltpu.SemaphoreType.DMA((QC,)),
            pltpu.SemaphoreType.DMA((QC,)),
            pltpu.SemaphoreType.DMA((QC,)),
            pltpu.SemaphoreType.DMA((QC,)),
            pltpu.SemaphoreType.DMA((QC,)),
            pltpu.SemaphoreType.DMA((QC,)),
            pltpu.SemaphoreType.DMA((2,)),
            pltpu.SemaphoreType.DMA((2,)),
            pltpu.SemaphoreType.DMA((2,)),
            pltpu.SemaphoreType.DMA((2,)),
        ],
        compiler_params=pltpu.CompilerParams(collective_id=0),
    )(partial, resid, gamma.reshape(1, d))
